# baseline (device time: 190546 ns/iter reference)
import jax
import jax.numpy as jnp
from jax import lax
from jax.experimental import pallas as pl
from jax.experimental.pallas import tpu as pltpu

N_DEV = 16
M_PER = 256
K = 4096
N_PER = 512

PARTS = 4
M_PART = M_PER // PARTS
N_MSG = PARTS * (N_DEV - 1) // 2

PERM = [0, 4, 8, 12, 15, 11, 7, 3, 2, 6, 10, 14, 13, 9, 5, 1]
INV = [0] * N_DEV
for _k, _l in enumerate(PERM):
    INV[_l] = _k
RIGHT = [PERM[(INV[_l] + 1) % N_DEV] for _l in range(N_DEV)]
LEFT = [PERM[(INV[_l] - 1) % N_DEV] for _l in range(N_DEV)]


def _lut(table, idx):
    r = jnp.int32(table[0])
    for v in range(1, len(table)):
        r = jnp.where(idx == v, jnp.int32(table[v]), r)
    return r


def kernel(x, w_mat):
    def body(x_ref, w_ref, out_ref, gather_ref, w_bf_ref,
             cw_send_sems, cw_recv_sems, ccw_send_sems, ccw_recv_sems):
        my = lax.axis_index("i")
        pos = _lut(INV, my)
        right = _lut(RIGHT, my)
        left = _lut(LEFT, my)

        barrier_sem = pltpu.get_barrier_semaphore()
        for nbr in (left, right):
            pl.semaphore_signal(
                barrier_sem, inc=1,
                device_id=(nbr,), device_id_type=pl.DeviceIdType.MESH,
            )
        pl.semaphore_wait(barrier_sem, 2)

        def _rdma(src, dst, ssem, rsem, target):
            return pltpu.make_async_remote_copy(
                src_ref=src, dst_ref=dst, send_sem=ssem, recv_sem=rsem,
                device_id=(target,), device_id_type=pl.DeviceIdType.MESH,
            )

        def cw_desc(j, target):
            s, q = j // PARTS, j % PARTS
            return _rdma(
                gather_ref.at[s, pl.ds(q * M_PART, M_PART), :],
                gather_ref.at[s + 1, pl.ds(q * M_PART, M_PART), :],
                cw_send_sems.at[j], cw_recv_sems.at[j], target,
            )

        def ccw_desc(j, target):
            s, q = j // PARTS, PARTS - 1 - (j % PARTS)
            return _rdma(
                gather_ref.at[(N_DEV - s) % N_DEV, pl.ds(q * M_PART, M_PART), :],
                gather_ref.at[N_DEV - 1 - s, pl.ds(q * M_PART, M_PART), :],
                ccw_send_sems.at[j], ccw_recv_sems.at[j], target,
            )

        def compute_part(slot, q):
            origin = _lut(PERM, (pos - slot) % N_DEV)
            acc = jnp.dot(
                gather_ref[slot, pl.ds(q * M_PART, M_PART), :],
                w_bf_ref[:, :],
                preferred_element_type=jnp.float32,
            )
            row = origin * M_PER + q * M_PART
            out_ref[pl.ds(row, M_PART), :] = jnp.maximum(acc, 0.0)

        sends = []

        def _cast_part(q):
            gather_ref[0, pl.ds(q * M_PART, M_PART), :] = (
                x_ref[pl.ds(q * M_PART, M_PART), :].astype(jnp.bfloat16)
            )

        _cast_part(0)
        d = cw_desc(0, right)
        d.start()
        sends.append(d)
        _cast_part(3)
        d = ccw_desc(0, left)
        d.start()
        sends.append(d)
        _cast_part(1)
        _cast_part(2)
        for j in range(1, PARTS):
            d = cw_desc(j, right)
            d.start()
            sends.append(d)
            d = ccw_desc(j, left)
            d.start()
            sends.append(d)

        w_bf_ref[:, :] = w_ref[:, :].astype(jnp.bfloat16)
        acc = jnp.dot(
            gather_ref[0, :, :], w_bf_ref[:, :],
            preferred_element_type=jnp.float32,
        )
        out_ref[pl.ds(my * M_PER, M_PER), :] = jnp.maximum(acc, 0.0)

        for j in range(N_MSG):
            cw_desc(j, left).wait_recv()
            if j + PARTS < N_MSG:
                d = cw_desc(j + PARTS, right)
                d.start()
                sends.append(d)
            ccw_desc(j, right).wait_recv()
            if j + PARTS < N_MSG:
                d = ccw_desc(j + PARTS, left)
                d.start()
                sends.append(d)
            compute_part(j // PARTS + 1, j % PARTS)
            compute_part(N_DEV - 1 - j // PARTS, PARTS - 1 - j % PARTS)

        for d in sends:
            d.wait_send()

    out_shape = jax.ShapeDtypeStruct((N_DEV * M_PER, N_PER), jnp.float32)
    return pl.pallas_call(
        body,
        out_shape=out_shape,
        in_specs=[
            pl.BlockSpec(memory_space=pltpu.VMEM),
            pl.BlockSpec(memory_space=pltpu.VMEM),
        ],
        out_specs=pl.BlockSpec(memory_space=pltpu.VMEM),
        scratch_shapes=[
            pltpu.VMEM((N_DEV, M_PER, K), jnp.bfloat16),
            pltpu.VMEM((K, N_PER), jnp.bfloat16),
            pltpu.SemaphoreType.DMA((N_MSG,)),
            pltpu.SemaphoreType.DMA((N_MSG,)),
            pltpu.SemaphoreType.DMA((N_MSG,)),
            pltpu.SemaphoreType.DMA((N_MSG,)),
        ],
        compiler_params=pltpu.CompilerParams(
            collective_id=0,
            vmem_limit_bytes=100 * 1024 * 1024,
        ),
    )(x, w_mat)


# device time: 190127 ns/iter; 1.0022x vs baseline; 1.0022x over previous
import jax
import jax.numpy as jnp
from jax import lax
from jax.experimental import pallas as pl
from jax.experimental.pallas import tpu as pltpu

N_DEV = 16
M_PER = 256
M_HALF = M_PER // 2
M_QTR = M_PER // 4
K = 4096
N_PER = 512

CW_MSGS = [(j // 2, (j % 2) * M_HALF, M_HALF) for j in range(14)] + [
    (7, 0, M_QTR),
    (7, M_QTR, M_QTR),
]
CCW_MSGS = [(j // 2, (1 - j % 2) * M_HALF, M_HALF) for j in range(14)] + [
    (7, M_HALF + M_QTR, M_QTR),
    (7, M_HALF, M_QTR),
]
N_MSG = len(CW_MSGS)

DEP = {m: m - 2 for m in range(2, 14)}
DEP[14] = 12
DEP[15] = 12
STARTS = {j: [m for m, d in DEP.items() if d == j] for j in range(N_MSG)}

PERM = [0, 4, 8, 12, 15, 11, 7, 3, 2, 6, 10, 14, 13, 9, 5, 1]
INV = [0] * N_DEV
for _k, _l in enumerate(PERM):
    INV[_l] = _k
RIGHT = [PERM[(INV[_l] + 1) % N_DEV] for _l in range(N_DEV)]
LEFT = [PERM[(INV[_l] - 1) % N_DEV] for _l in range(N_DEV)]


def _lut(table, idx):
    r = jnp.int32(table[0])
    for v in range(1, len(table)):
        r = jnp.where(idx == v, jnp.int32(table[v]), r)
    return r


def kernel(x, w_mat):
    def body(x_ref, w_ref, out_ref, gather_ref, w_bf_ref,
             cw_send_sems, cw_recv_sems, ccw_send_sems, ccw_recv_sems):
        my = lax.axis_index("i")
        pos = _lut(INV, my)
        right = _lut(RIGHT, my)
        left = _lut(LEFT, my)

        barrier_sem = pltpu.get_barrier_semaphore()
        for nbr in (left, right):
            pl.semaphore_signal(
                barrier_sem, inc=1,
                device_id=(nbr,), device_id_type=pl.DeviceIdType.MESH,
            )
        pl.semaphore_wait(barrier_sem, 2)

        def _rdma(src, dst, ssem, rsem, target):
            return pltpu.make_async_remote_copy(
                src_ref=src, dst_ref=dst, send_sem=ssem, recv_sem=rsem,
                device_id=(target,), device_id_type=pl.DeviceIdType.MESH,
            )

        def cw_desc(m, target):
            s, r0, nr = CW_MSGS[m]
            return _rdma(
                gather_ref.at[s, pl.ds(r0, nr), :],
                gather_ref.at[s + 1, pl.ds(r0, nr), :],
                cw_send_sems.at[m], cw_recv_sems.at[m], target,
            )

        def ccw_desc(m, target):
            s, r0, nr = CCW_MSGS[m]
            return _rdma(
                gather_ref.at[(N_DEV - s) % N_DEV, pl.ds(r0, nr), :],
                gather_ref.at[N_DEV - 1 - s, pl.ds(r0, nr), :],
                ccw_send_sems.at[m], ccw_recv_sems.at[m], target,
            )

        def compute_rows(slot, r0, nr):
            origin = _lut(PERM, (pos - slot) % N_DEV)
            acc = jnp.dot(
                gather_ref[slot, pl.ds(r0, nr), :],
                w_bf_ref[:, :],
                preferred_element_type=jnp.float32,
            )
            out_ref[pl.ds(origin * M_PER + r0, nr), :] = jnp.maximum(acc, 0.0)

        sends = []

        def _start(d):
            d.start()
            sends.append(d)

        gather_ref[0, pl.ds(0, M_HALF), :] = (
            x_ref[pl.ds(0, M_HALF), :].astype(jnp.bfloat16))
        _start(cw_desc(0, right))
        gather_ref[0, pl.ds(M_HALF, M_HALF), :] = (
            x_ref[pl.ds(M_HALF, M_HALF), :].astype(jnp.bfloat16))
        _start(ccw_desc(0, left))
        _start(cw_desc(1, right))
        _start(ccw_desc(1, left))

        w_bf_ref[:, :] = w_ref[:, :].astype(jnp.bfloat16)
        compute_rows(0, 0, M_PER)

        for j in range(N_MSG):
            cw_desc(j, left).wait_recv()
            for m in STARTS.get(j, ()):
                _start(cw_desc(m, right))
            ccw_desc(j, right).wait_recv()
            for m in STARTS.get(j, ()):
                _start(ccw_desc(m, left))
            s, r0, nr = CW_MSGS[j]
            compute_rows(s + 1, r0, nr)
            s, r0, nr = CCW_MSGS[j]
            compute_rows(N_DEV - 1 - s, r0, nr)

        for d in sends:
            d.wait_send()

    out_shape = jax.ShapeDtypeStruct((N_DEV * M_PER, N_PER), jnp.float32)
    return pl.pallas_call(
        body,
        out_shape=out_shape,
        in_specs=[
            pl.BlockSpec(memory_space=pltpu.VMEM),
            pl.BlockSpec(memory_space=pltpu.VMEM),
        ],
        out_specs=pl.BlockSpec(memory_space=pltpu.VMEM),
        scratch_shapes=[
            pltpu.VMEM((N_DEV, M_PER, K), jnp.bfloat16),
            pltpu.VMEM((K, N_PER), jnp.bfloat16),
            pltpu.SemaphoreType.DMA((N_MSG,)),
            pltpu.SemaphoreType.DMA((N_MSG,)),
            pltpu.SemaphoreType.DMA((N_MSG,)),
            pltpu.SemaphoreType.DMA((N_MSG,)),
        ],
        compiler_params=pltpu.CompilerParams(
            collective_id=0,
            vmem_limit_bytes=100 * 1024 * 1024,
        ),
    )(x, w_mat)
